# baseline (speedup 1.0000x reference)
import functools

import jax
import jax.numpy as jnp
from jax import lax
from jax.experimental import pallas as pl
from jax.experimental.pallas import tpu as pltpu
from jax.experimental.pallas import tpu_sc as plsc

D_MODEL = 64
SCALE = 8.0
LANES = 16

NC, NS = 2, 16
NW = NC * NS
B_TOTAL = 4096 * 200
IDXW = 128
ROWS_TOTAL = B_TOTAL // IDXW
ROWS_PER_W = ROWS_TOTAL // NW
CH_ROWS = 4
CH = CH_ROWS * IDXW
N_CH = ROWS_PER_W // CH_ROWS

@functools.cache
def _build_embed_sc():
    mesh = plsc.VectorSubcoreMesh(core_axis_name="c", subcore_axis_name="s")
    return pl.kernel(
        _embed_sc_body,
        out_type=jax.ShapeDtypeStruct((B_TOTAL, D_MODEL), jnp.float32),
        mesh=mesh,
        scratch_types=[
            pltpu.VMEM((CH_ROWS, IDXW), jnp.int32),
            pltpu.VMEM((CH, D_MODEL), jnp.float32),
            pltpu.SemaphoreType.DMA,
        ],
        compiler_params=pltpu.CompilerParams(use_tc_tiling_on_sc=False),
    )


def _embed_sc_body(x_hbm, table_hbm, out_hbm, idx_v, rows_v, sem):
    wid = lax.axis_index("s") * NC + lax.axis_index("c")
    row_base = wid * ROWS_PER_W
    out_base = wid * (ROWS_PER_W * IDXW)

    def chunk_body(g, carry):
        pltpu.sync_copy(x_hbm.at[pl.ds(row_base + g * CH_ROWS, CH_ROWS)], idx_v)
        copies = [
            pltpu.async_copy(
                table_hbm.at[idx_v.at[j]],
                rows_v.at[pl.ds(j * IDXW, IDXW)],
                sem,
            )
            for j in range(CH_ROWS)
        ]
        for c in copies:
            c.wait()

        def scale_row(r, acc):
            for dd in range(D_MODEL // LANES):
                sl = pl.ds(dd * LANES, LANES)
                rows_v[r, sl] = rows_v[r, sl] * SCALE
            return acc

        lax.fori_loop(0, CH, scale_row, 0)

        pltpu.sync_copy(rows_v, out_hbm.at[pl.ds(out_base + g * CH, CH)])
        return carry

    lax.fori_loop(0, N_CH, chunk_body, 0)


def kernel(x, table):
    xf = x.astype(jnp.int32).reshape(ROWS_TOTAL, IDXW)
    out = _build_embed_sc()(xf, table)
    return out.reshape(x.shape[0], x.shape[1], D_MODEL)


# speedup vs baseline: 1.1213x; 1.1213x over previous
import functools

import jax
import jax.numpy as jnp
from jax import lax
from jax.experimental import pallas as pl
from jax.experimental.pallas import tpu as pltpu
from jax.experimental.pallas import tpu_sc as plsc

D_MODEL = 64
SCALE = 8.0
LANES = 16

NC, NS = 2, 16
NW = NC * NS
B_TOTAL = 4096 * 200
IDXW = 128
ROWS_TOTAL = B_TOTAL // IDXW
ROWS_PER_W = ROWS_TOTAL // NW
CH = IDXW
N_CH = ROWS_PER_W
NBUF = 4


@functools.cache
def _build_embed_sc():
    mesh = plsc.VectorSubcoreMesh(core_axis_name="c", subcore_axis_name="s")
    return pl.kernel(
        _embed_sc_body,
        out_type=jax.ShapeDtypeStruct((B_TOTAL, D_MODEL), jnp.float32),
        mesh=mesh,
        scratch_types=[
            pltpu.VMEM((ROWS_PER_W, IDXW), jnp.int32),
            pltpu.VMEM((NBUF * CH, D_MODEL), jnp.float32),
            [pltpu.SemaphoreType.DMA] * NBUF,
            [pltpu.SemaphoreType.DMA] * NBUF,
        ],
        compiler_params=pltpu.CompilerParams(use_tc_tiling_on_sc=False),
    )


def _embed_sc_body(x_hbm, table_hbm, out_hbm, idx_v, rows_v, gsem, ssem):
    wid = lax.axis_index("s") * NC + lax.axis_index("c")
    row_base = wid * ROWS_PER_W
    out_base = wid * (ROWS_PER_W * CH)

    pltpu.sync_copy(x_hbm.at[pl.ds(row_base, ROWS_PER_W)], idx_v)

    def gather(h, b):
        return pltpu.make_async_copy(
            table_hbm.at[idx_v.at[h]],
            rows_v.at[pl.ds(b * CH, CH)],
            gsem[b],
        )

    def store(h, b):
        return pltpu.make_async_copy(
            rows_v.at[pl.ds(b * CH, CH)],
            out_hbm.at[pl.ds(out_base + h * CH, CH)],
            ssem[b],
        )

    gather(0, 0).start()
    gather(1, 1).start()

    @pl.loop(0, N_CH, step=NBUF)
    def _chunks(g0):
        for b in range(NBUF):
            g = g0 + b
            b2 = (b + 2) % NBUF
            gather(g, b).wait()

            @plsc.parallel_loop(0, CH, unroll=4)
            def _scale(r):
                row = b * CH + r
                for dd in range(D_MODEL // LANES):
                    sl = pl.ds(dd * LANES, LANES)
                    rows_v[row, sl] = rows_v[row, sl] * SCALE

            store(g, b).start()

            @pl.when(g >= 2)
            def _drain_old_store():
                store(g - 2, b2).wait()

            @pl.when(g + 2 < N_CH)
            def _prefetch():
                gather(g + 2, b2).start()

    store(N_CH - 2, (N_CH - 2) % NBUF).wait()
    store(N_CH - 1, (N_CH - 1) % NBUF).wait()


def kernel(x, table):
    xf = x.astype(jnp.int32).reshape(ROWS_TOTAL, IDXW)
    out = _build_embed_sc()(xf, table)
    return out.reshape(x.shape[0], x.shape[1], D_MODEL)


# speedup vs baseline: 1.1625x; 1.0368x over previous
import functools

import jax
import jax.numpy as jnp
from jax import lax
from jax.experimental import pallas as pl
from jax.experimental.pallas import tpu as pltpu
from jax.experimental.pallas import tpu_sc as plsc

D_MODEL = 64
SCALE = 8.0
L = 16

NC, NS = 2, 16
NW = NC * NS
S, T = 4096, 200
B_TOTAL = S * T
SPW = S // NW
IPW = SPW * T
VPAD = 128
NBUF = 4
NDT = D_MODEL // 8


@functools.cache
def _build_embed_sc():
    mesh = plsc.VectorSubcoreMesh(core_axis_name="c", subcore_axis_name="s")
    return pl.kernel(
        _embed_sc_body,
        out_type=jax.ShapeDtypeStruct((T, NDT, NW, 8, SPW), jnp.float32),
        mesh=mesh,
        scratch_types=[
            pltpu.VMEM((IPW,), jnp.int32),
            pltpu.VMEM((NBUF, SPW), jnp.int32),
            pltpu.VMEM((NBUF * SPW, VPAD), jnp.float32),
            pltpu.VMEM((2, NDT, 8, SPW), jnp.float32),
            [pltpu.SemaphoreType.DMA] * NBUF,
            [pltpu.SemaphoreType.DMA] * 2,
        ],
        compiler_params=pltpu.CompilerParams(needs_layout_passes=False),
    )


def _embed_sc_body(xf, tpad, out5, slab, idxr, rows, oblk, gsem, ssem):
    w = lax.axis_index("s") * NC + lax.axis_index("c")
    pltpu.sync_copy(xf.at[pl.ds(w * IPW, IPW)], slab)
    iota = lax.iota(jnp.int32, L)
    uvec = [(jnp.int32(sv * L) + iota) * T for sv in range(SPW // L)]
    svec = [jnp.int32(sv * L) + iota for sv in range(SPW // L)]

    def build_idx(t, b):
        for sv in range(SPW // L):
            idxr[b, pl.ds(sv * L, L)] = plsc.load_gather(slab, [uvec[sv] + t])

    def gather(b):
        return pltpu.make_async_copy(
            tpad.at[idxr.at[b]], rows.at[pl.ds(b * SPW, SPW)], gsem[b]
        )

    def store(t, ob, dt):
        return pltpu.make_async_copy(
            oblk.at[ob, dt], out5.at[t, dt, w], ssem[ob]
        )

    build_idx(0, 0)
    gather(0).start()
    build_idx(1, 1)
    gather(1).start()

    @pl.loop(0, T, step=NBUF)
    def _t_loop(g0):
        for b in range(NBUF):
            ob = b % 2
            b2 = (b + 2) % NBUF
            t = g0 + b
            gather(b).wait()

            @pl.when(t >= 2)
            def _drain_stores():
                for dt in range(NDT):
                    store(t - 2, ob, dt).wait()

            @plsc.parallel_loop(0, D_MODEL, unroll=2)
            def _transpose_scale(c):
                col = jnp.broadcast_to(c, (L,))
                for sv in range(SPW // L):
                    v = plsc.load_gather(
                        rows, [svec[sv] + jnp.int32(b * SPW), col]
                    )
                    oblk[ob, c >> 3, c & 7, pl.ds(sv * L, L)] = v * SCALE

            for dt in range(NDT):
                store(t, ob, dt).start()

            @pl.when(t + 2 < T)
            def _prefetch():
                build_idx(t + 2, b2)
                gather(b2).start()

    for t in (T - 2, T - 1):
        for dt in range(NDT):
            store(t, t % 2, dt).wait()


def kernel(x, table):
    xf = x.astype(jnp.int32).reshape(-1)
    tpad = jnp.pad(table, ((0, 0), (0, VPAD - D_MODEL)))
    out5 = _build_embed_sc()(xf, tpad)
    return out5.transpose(2, 4, 0, 1, 3).reshape(S, T, D_MODEL)
